# baseline (speedup 1.0000x reference)
import functools

import jax
import jax.numpy as jnp
from jax.experimental import pallas as pl
from jax.experimental.pallas import tpu as pltpu
from jax.experimental.pallas import tpu_sc as plsc

_E = 8
_TOP_K = 2
_T = 32
_H = 2048
_I = 4096
_TS = 512
_NI = _I // _TS


_SC_CORES = 1
_SC_SUBCORES = 16
_SC_LANES = 16


def _make_routing_sc():
    nc, nl = _SC_CORES, _SC_LANES

    mesh = plsc.VectorSubcoreMesh(
        core_axis_name="c", subcore_axis_name="s",
        num_cores=_SC_CORES, num_subcores=_SC_SUBCORES)

    @functools.partial(
        pl.kernel, mesh=mesh,
        out_type=jax.ShapeDtypeStruct((_T * _E,), jnp.float32),
        compiler_params=pltpu.CompilerParams(needs_layout_passes=False),
        scratch_types=[
            pltpu.VMEM((nl,), jnp.int32),
            pltpu.VMEM((nl,), jnp.float32),
            pltpu.VMEM((nl,), jnp.float32),
        ],
    )
    def routing_sc(idx_hbm, aff_hbm, out_hbm, idx_v, aff_v, w_v):
        s = jax.lax.axis_index("s")
        chunk = s // 4
        q = (s % 4) * 2 * _TOP_K
        pltpu.sync_copy(idx_hbm.at[pl.ds(chunk * nl, nl)], idx_v)
        pltpu.sync_copy(aff_hbm.at[pl.ds(s * nl, nl)], aff_v)

        v = idx_v[...]
        lane = jax.lax.iota(jnp.int32, nl)
        eid = lane % _E
        low = lane < _E
        a0 = jnp.sum(jnp.where(lane == q, v, 0))
        a1 = jnp.sum(jnp.where(lane == q + 1, v, 0))
        b0 = jnp.sum(jnp.where(lane == q + 2, v, 0))
        b1 = jnp.sum(jnp.where(lane == q + 3, v, 0))
        chosen = jnp.where(low, (eid == a0) | (eid == a1),
                           (eid == b0) | (eid == b1))
        m = jnp.where(chosen, aff_v[...], 0.0)
        am = jnp.abs(m)
        tot = jnp.sum(am)
        s0 = jnp.sum(jnp.where(low, am, 0.0))
        denom = jnp.maximum(jnp.where(low, s0, tot - s0), 1e-12)
        w_v[...] = m / denom
        pltpu.sync_copy(w_v, out_hbm.at[pl.ds(s * nl, nl)])

    return routing_sc


_routing_sc = _make_routing_sc()


def _mlp_kernel(w_ref, x_ref, gate_ref, up_ref, down_ref, out_ref):
    e = pl.program_id(0)
    i = pl.program_id(1)

    w = w_ref[:, :]
    ecol = jax.lax.broadcasted_iota(jnp.int32, w.shape, 1)
    we = jnp.sum(jnp.where(ecol == e, w, 0.0), axis=1, keepdims=True)

    x = x_ref[:, :]
    g = jnp.dot(x, gate_ref[0], preferred_element_type=jnp.float32)
    u = jnp.dot(x, up_ref[0], preferred_element_type=jnp.float32)
    inter = (g * jax.nn.sigmoid(g)) * u * we
    contrib = jnp.dot(inter, down_ref[0], preferred_element_type=jnp.float32)

    @pl.when((e == 0) & (i == 0))
    def _init():
        out_ref[:, :] = jnp.zeros_like(out_ref)

    out_ref[:, :] += contrib


@functools.partial(jax.jit, static_argnames=())
def kernel(hidden_states, expert_affinities, expert_index, gate_up_proj, down_proj):
    idx_flat = expert_index.astype(jnp.int32).reshape(-1)
    w_flat = _routing_sc(idx_flat, expert_affinities.reshape(-1))
    w = w_flat.reshape(_T, _E)

    grid = (_E, _NI)
    return pl.pallas_call(
        _mlp_kernel,
        grid=grid,
        in_specs=[
            pl.BlockSpec((_T, _E), lambda e, i: (0, 0)),
            pl.BlockSpec((_T, _H), lambda e, i: (0, 0)),
            pl.BlockSpec((1, _H, _TS), lambda e, i: (e, 0, i)),
            pl.BlockSpec((1, _H, _TS), lambda e, i: (e, 0, _NI + i)),
            pl.BlockSpec((1, _TS, _H), lambda e, i: (e, i, 0)),
        ],
        out_specs=pl.BlockSpec((_T, _H), lambda e, i: (0, 0)),
        out_shape=jax.ShapeDtypeStruct((_T, _H), jnp.float32),
    )(w, hidden_states, gate_up_proj, gate_up_proj, down_proj)


# speedup vs baseline: 1.0307x; 1.0307x over previous
import functools

import jax
import jax.numpy as jnp
from jax.experimental import pallas as pl
from jax.experimental.pallas import tpu as pltpu
from jax.experimental.pallas import tpu_sc as plsc

_E = 8
_TOP_K = 2
_T = 32
_H = 2048
_I = 4096
_TS = 512
_NI = _I // _TS

_SC_CORES = 2
_SC_SUBCORES = 16
_SC_LANES = 16


def _make_routing_sc():
    nc, nl = _SC_CORES, _SC_LANES

    mesh = plsc.VectorSubcoreMesh(
        core_axis_name="c", subcore_axis_name="s",
        num_cores=_SC_CORES, num_subcores=_SC_SUBCORES)

    @functools.partial(
        pl.kernel, mesh=mesh,
        out_type=jax.ShapeDtypeStruct((_T * _E,), jnp.float32),
        compiler_params=pltpu.CompilerParams(needs_layout_passes=False),
        scratch_types=[
            pltpu.VMEM((nl,), jnp.int32),
            pltpu.VMEM((nl,), jnp.float32),
            pltpu.VMEM((nl,), jnp.float32),
        ],
    )
    def routing_sc(idx_hbm, aff_hbm, out_hbm, idx_v, aff_v, w_v):
        tok = jax.lax.axis_index("s") * nc + jax.lax.axis_index("c")

        @pl.when(tok < _T)
        def _():
            chunk = tok // (nl // _TOP_K)
            p0 = _TOP_K * (tok % (nl // _TOP_K))
            pltpu.sync_copy(idx_hbm.at[pl.ds(chunk * nl, nl)], idx_v)
            r = tok % 2
            pltpu.sync_copy(aff_hbm.at[pl.ds((tok - r) * _E, nl)], aff_v)

            v = idx_v[...]
            lane = jax.lax.iota(jnp.int32, nl)
            eid = lane % _E
            mine = (lane // _E) == r
            i0 = jnp.sum(jnp.where(lane == p0, v, 0))
            i1 = jnp.sum(jnp.where(lane == p0 + 1, v, 0))
            chosen = ((i0 == eid) | (i1 == eid)) & mine
            a = aff_v[...]
            m = jnp.where(chosen, a, 0.0)
            denom = jnp.maximum(jnp.sum(jnp.abs(m)), 1e-12)
            w_v[...] = m / denom
            pltpu.sync_copy(w_v.at[pl.ds(r * _E, _E)],
                            out_hbm.at[pl.ds(tok * _E, _E)])

    return routing_sc


_routing_sc = _make_routing_sc()


def _mlp_kernel(x_ref, gate_ref, up_ref, down_ref, out_ref):
    i = pl.program_id(1)

    x = x_ref[:, :]
    g = jnp.dot(x, gate_ref[0], preferred_element_type=jnp.float32)
    u = jnp.dot(x, up_ref[0], preferred_element_type=jnp.float32)
    inter = (g * jax.nn.sigmoid(g)) * u
    contrib = jnp.dot(inter, down_ref[0], preferred_element_type=jnp.float32)

    @pl.when(i == 0)
    def _init():
        out_ref[0] = contrib

    @pl.when(i > 0)
    def _acc():
        out_ref[0] += contrib


def _combine_kernel(w_ref, y_ref, out_ref):
    w = w_ref[:, :]
    ecol = jax.lax.broadcasted_iota(jnp.int32, w.shape, 1)
    acc = jnp.zeros((_T, _H), jnp.float32)
    for e in range(_E):
        we = jnp.sum(jnp.where(ecol == e, w, 0.0), axis=1, keepdims=True)
        acc += y_ref[e] * we
    out_ref[:, :] = acc


@functools.partial(jax.jit, static_argnames=())
def kernel(hidden_states, expert_affinities, expert_index, gate_up_proj, down_proj):
    idx_flat = expert_index.astype(jnp.int32).reshape(-1)
    w_flat = _routing_sc(idx_flat, expert_affinities.reshape(-1))
    w = w_flat.reshape(_T, _E)

    y = pl.pallas_call(
        _mlp_kernel,
        grid=(_E, _NI),
        in_specs=[
            pl.BlockSpec((_T, _H), lambda e, i: (0, 0)),
            pl.BlockSpec((1, _H, _TS), lambda e, i: (e, 0, i)),
            pl.BlockSpec((1, _H, _TS), lambda e, i: (e, 0, _NI + i)),
            pl.BlockSpec((1, _TS, _H), lambda e, i: (e, i, 0)),
        ],
        out_specs=pl.BlockSpec((1, _T, _H), lambda e, i: (e, 0, 0)),
        out_shape=jax.ShapeDtypeStruct((_E, _T, _H), jnp.float32),
    )(hidden_states, gate_up_proj, gate_up_proj, down_proj)

    return pl.pallas_call(
        _combine_kernel,
        in_specs=[
            pl.BlockSpec((_T, _E), lambda: (0, 0)),
            pl.BlockSpec((_E, _T, _H), lambda: (0, 0, 0)),
        ],
        out_specs=pl.BlockSpec((_T, _H), lambda: (0, 0)),
        out_shape=jax.ShapeDtypeStruct((_T, _H), jnp.float32),
    )(w, y)
